# baseline (speedup 1.0000x reference)
import functools

import jax
import jax.numpy as jnp
from jax import lax
from jax.experimental import pallas as pl
from jax.experimental.pallas import tpu as pltpu
from jax.experimental.pallas import tpu_sc as plsc

NC = 2
NS = 16
NW = NC * NS
EB = 2000


def _mesh():
    return plsc.VectorSubcoreMesh(core_axis_name="c", subcore_axis_name="s")


_SC_PARAMS = pltpu.CompilerParams(use_tc_tiling_on_sc=False)


def _make_degree(NP, E):
    per_tile = E // NW
    n_blk = per_tile // EB
    z = NP // NS

    @functools.partial(
        pl.kernel,
        out_type=jax.ShapeDtypeStruct((NC * NP,), jnp.float32),
        mesh=_mesh(),
        compiler_params=_SC_PARAMS,
        scratch_types=[
            pltpu.VMEM((EB,), jnp.int32),
            pltpu.VMEM((EB,), jnp.float32),
            pltpu.VMEM((z,), jnp.float32),
            pltpu.VMEM_SHARED((NP,), jnp.float32),
            pltpu.SemaphoreType.DMA,
        ],
    )
    def deg_kernel(dst_hbm, out_hbm, didx, ones_v, stage, acc, sem):
        c = lax.axis_index("c")
        s = lax.axis_index("s")
        tile_base = (c * NS + s) * per_tile

        def set_ones(i, _):
            ones_v[pl.ds(i * 16, 16)] = jnp.full((16,), 1.0, jnp.float32)
            return 0

        lax.fori_loop(0, EB // 16, set_ones, 0)

        def set_zero(i, _):
            stage[pl.ds(i * 16, 16)] = jnp.zeros((16,), jnp.float32)
            return 0

        lax.fori_loop(0, z // 16, set_zero, 0)
        pltpu.sync_copy(stage, acc.at[pl.ds(s * z, z)])
        plsc.subcore_barrier()

        def blk(i, _):
            base = tile_base + i * EB
            pltpu.sync_copy(dst_hbm.at[pl.ds(base, EB)], didx)
            pltpu.sync_copy(ones_v, acc.at[didx], add=True)
            return 0

        lax.fori_loop(0, n_blk, blk, 0)
        plsc.subcore_barrier()
        pltpu.sync_copy(acc.at[pl.ds(s * z, z)], stage)
        pltpu.sync_copy(stage, out_hbm.at[pl.ds(c * NP + s * z, z)])

    return deg_kernel


def _make_prop16(NP, E):
    EB = 400
    per_tile = E // NW
    n_blk = per_tile // EB
    z = NP // NS

    @functools.partial(
        pl.kernel,
        out_type=jax.ShapeDtypeStruct((NC * NP, 16), jnp.float32),
        mesh=_mesh(),
        compiler_params=_SC_PARAMS,
        scratch_types=[
            pltpu.VMEM((EB,), jnp.int32),
            pltpu.VMEM((EB,), jnp.int32),
            pltpu.VMEM((EB, 16), jnp.float32),
            pltpu.VMEM_SHARED((NP, 16), jnp.float32),
            pltpu.SemaphoreType.DMA,
        ],
    )
    def prop_kernel(y_hbm, src_hbm, dst_hbm, out_hbm,
                    sidx, didx, rows, acc, sem):
        c = lax.axis_index("c")
        s = lax.axis_index("s")
        tile_base = (c * NS + s) * per_tile
        chunks = []
        off = 0
        while off < z:
            n = min(EB, z - off)
            chunks.append((off, n))
            off += n

        def fill_zero(i, _):
            rows[i] = jnp.zeros((16,), jnp.float32)
            return 0

        lax.fori_loop(0, EB, fill_zero, 0)
        for (o, n) in chunks:
            pltpu.sync_copy(rows.at[pl.ds(0, n)], acc.at[pl.ds(s * z + o, n)])
        plsc.subcore_barrier()

        def blk(i, _):
            base = tile_base + i * EB
            pltpu.sync_copy(src_hbm.at[pl.ds(base, EB)], sidx)
            pltpu.sync_copy(dst_hbm.at[pl.ds(base, EB)], didx)
            pltpu.async_copy(y_hbm.at[sidx], rows, sem).wait()
            pltpu.sync_copy(rows, acc.at[didx], add=True)
            return 0

        lax.fori_loop(0, n_blk, blk, 0)
        plsc.subcore_barrier()
        for (o, n) in chunks:
            pltpu.sync_copy(acc.at[pl.ds(s * z + o, n)], rows.at[pl.ds(0, n)])
            pltpu.sync_copy(rows.at[pl.ds(0, n)],
                            out_hbm.at[pl.ds(c * NP + s * z + o, n)])

    return prop_kernel


def _make_prop64(NP, E):
    EB = 400
    per_tile = E // NS
    n_blk = per_tile // EB
    z = NP // NS

    @functools.partial(
        pl.kernel,
        out_type=jax.ShapeDtypeStruct((4 * NP, 16), jnp.float32),
        mesh=_mesh(),
        compiler_params=_SC_PARAMS,
        scratch_types=[
            pltpu.VMEM((EB,), jnp.int32),
            pltpu.VMEM((EB,), jnp.int32),
            pltpu.VMEM((EB,), jnp.int32),
            pltpu.VMEM((EB,), jnp.int32),
            pltpu.VMEM((EB, 16), jnp.float32),
            pltpu.VMEM((EB, 16), jnp.float32),
            pltpu.SemaphoreType.DMA,
            pltpu.SemaphoreType.DMA,
            pltpu.VMEM_SHARED((NP, 16), jnp.float32),
        ],
    )
    def prop_kernel(u0_hbm, u1_hbm, u2_hbm, u3_hbm, src_hbm, dst_hbm,
                    out_hbm, sidx0, sidx1, didx0, didx1, rows0, rows1,
                    sem0, sem1, acc):
        c = lax.axis_index("c")
        s = lax.axis_index("s")
        tile_base = s * per_tile
        u_refs = (u0_hbm, u1_hbm, u2_hbm, u3_hbm)
        sidx = (sidx0, sidx1)
        didx = (didx0, didx1)
        rows = (rows0, rows1)
        sems = (sem0, sem1)
        chunks = []
        off = 0
        while off < z:
            n = min(EB, z - off)
            chunks.append((off, n))
            off += n

        def fill_zero(i, _):
            rows0[i] = jnp.zeros((16,), jnp.float32)
            return 0

        def load_and_gather(b, blk_i, q):
            base = tile_base + blk_i * EB
            pltpu.sync_copy(src_hbm.at[pl.ds(base, EB)], sidx[b])
            pltpu.sync_copy(dst_hbm.at[pl.ds(base, EB)], didx[b])
            for qq in range(4):
                @pl.when(q == qq)
                def _gather():
                    pltpu.async_copy(u_refs[qq].at[sidx[b]], rows[b],
                                     sems[b])

        for r in range(2):
            q = c + 2 * r

            lax.fori_loop(0, EB, fill_zero, 0)
            for (o, n) in chunks:
                pltpu.sync_copy(rows0.at[pl.ds(0, n)],
                                acc.at[pl.ds(s * z + o, n)])
            plsc.subcore_barrier()

            load_and_gather(0, 0, q)

            def pair(i2, _):
                for b in range(2):
                    i = 2 * i2 + b
                    pltpu.make_async_copy(u0_hbm.at[sidx[b]], rows[b],
                                          sems[b]).wait()

                    @pl.when(i + 1 < n_blk)
                    def _prefetch():
                        load_and_gather(1 - b, i + 1, q)

                    pltpu.sync_copy(rows[b], acc.at[didx[b]], add=True)
                return 0

            lax.fori_loop(0, n_blk // 2, pair, 0)
            plsc.subcore_barrier()
            for (o, n) in chunks:
                pltpu.sync_copy(acc.at[pl.ds(s * z + o, n)],
                                rows0.at[pl.ds(0, n)])
                pltpu.sync_copy(rows0.at[pl.ds(0, n)],
                                out_hbm.at[pl.ds(q * NP + s * z + o, n)])
            plsc.subcore_barrier()

    return prop_kernel


_BN = 3128


def _tc_a(deg2, x):
    NP = x.shape[0]
    nb = NP // _BN

    def body(d0_r, d1_r, x_r, dinv_o, y_o):
        deg = d0_r[...] + d1_r[...] + 1.0
        dinv = lax.rsqrt(jnp.maximum(deg, 1.0))
        dinv_o[...] = dinv
        xb = x_r[...].astype(jnp.float32)
        y_o[...] = jnp.concatenate(
            [xb * dinv, jnp.zeros((_BN, 14), jnp.float32)], axis=1)

    return pl.pallas_call(
        body,
        grid=(nb,),
        in_specs=[
            pl.BlockSpec((_BN, 1), lambda i: (i, 0)),
            pl.BlockSpec((_BN, 1), lambda i: (i + NP // _BN, 0)),
            pl.BlockSpec((_BN, 2), lambda i: (i, 0)),
        ],
        out_specs=[
            pl.BlockSpec((_BN, 1), lambda i: (i, 0)),
            pl.BlockSpec((_BN, 16), lambda i: (i, 0)),
        ],
        out_shape=[
            jax.ShapeDtypeStruct((NP, 1), jnp.float32),
            jax.ShapeDtypeStruct((NP, 16), jnp.float32),
        ],
    )(deg2, deg2, x)


def _tc_b(zp, y16, dinv, W1, b1, W2):
    NP = y16.shape[0]
    nb = NP // _BN

    def body(z0_r, z1_r, y_r, dinv_r, w1_r, b1_r, w2_r,
             u0_o, u1_o, u2_o, u3_o):
        p = (z0_r[..., :2] + z1_r[..., :2] + y_r[..., :2]) * dinv_r[...]
        w1 = w1_r[...].astype(jnp.float32)
        h1 = jnp.maximum(
            p[:, 0:1] * w1[0:1, :] + p[:, 1:2] * w1[1:2, :] + b1_r[...], 0.0)
        t = jnp.dot(h1.astype(jnp.bfloat16), w2_r[...],
                    preferred_element_type=jnp.float32)
        u = t * dinv_r[...]
        for q, o_r in enumerate((u0_o, u1_o, u2_o, u3_o)):
            o_r[...] = u[:, 16 * q:16 * q + 16]

    qshape = jax.ShapeDtypeStruct((NP, 16), jnp.float32)
    return pl.pallas_call(
        body,
        grid=(nb,),
        in_specs=[
            pl.BlockSpec((_BN, 16), lambda i: (i, 0)),
            pl.BlockSpec((_BN, 16), lambda i: (i + NP // _BN, 0)),
            pl.BlockSpec((_BN, 16), lambda i: (i, 0)),
            pl.BlockSpec((_BN, 1), lambda i: (i, 0)),
            pl.BlockSpec((2, 128), lambda i: (0, 0)),
            pl.BlockSpec((1, 128), lambda i: (0, 0)),
            pl.BlockSpec((128, 64), lambda i: (0, 0)),
        ],
        out_specs=[pl.BlockSpec((_BN, 16), lambda i: (i, 0))] * 4,
        out_shape=[qshape] * 4,
    )(zp, zp, y16, dinv, W1, b1, W2)


def _tc_c(v4, u0, u1, u2, u3, dinv, b2, wpT, bp):
    NP = dinv.shape[0]
    nb = NP // _BN

    def qmap(q):
        return lambda i: (q * (NP // _BN) + i, 0)

    def body(v0, v1, v2, v3, u0, u1, u2, u3, dinv_r, b2_r, wp_r, bp_r, o_r):
        dinv = dinv_r[...]
        b2 = b2_r[...]
        wp = wp_r[...]
        acc = jnp.zeros((_BN, 1), jnp.float32) + bp_r[...]
        for q, (v_r, u_r) in enumerate(((v0, u0), (v1, u1), (v2, u2),
                                        (v3, u3))):
            h2 = jnp.maximum(
                (v_r[...] + u_r[...]) * dinv + b2[:, 16 * q:16 * q + 16], 0.0)
            acc = acc + jnp.dot(h2.astype(jnp.bfloat16),
                                wp[q].reshape(16, 1),
                                preferred_element_type=jnp.float32)
        o_r[...] = acc

    return pl.pallas_call(
        body,
        grid=(nb,),
        in_specs=[pl.BlockSpec((_BN, 16), qmap(q)) for q in range(4)]
        + [pl.BlockSpec((_BN, 16), lambda i: (i, 0)) for _ in range(4)]
        + [
            pl.BlockSpec((_BN, 1), lambda i: (i, 0)),
            pl.BlockSpec((1, 64), lambda i: (0, 0)),
            pl.BlockSpec((4, 16), lambda i: (0, 0)),
            pl.BlockSpec((1, 1), lambda i: (0, 0)),
        ],
        out_specs=pl.BlockSpec((_BN, 1), lambda i: (i, 0)),
        out_shape=jax.ShapeDtypeStruct((NP, 1), jnp.float32),
    )(v4, v4, v4, v4, u0, u1, u2, u3, dinv, b2, wpT, bp)


def kernel(x, edge_index, W1, b1, W2, b2, Wp, bp):
    N = x.shape[0]
    E = edge_index.shape[1]
    NP = ((N + 8 * NS - 1) // (8 * NS)) * (8 * NS)
    src = edge_index[0]
    dst = edge_index[1]

    x_p = jnp.pad(x, ((0, NP - N), (0, 0))).astype(jnp.bfloat16)

    deg2 = _make_degree(NP, E)(dst).reshape(2 * NP, 1)
    dinv, y16 = _tc_a(deg2, x_p)

    zp = _make_prop16(NP, E)(y16, src, dst)
    u0, u1, u2, u3 = _tc_b(zp, y16, dinv, W1.astype(jnp.bfloat16),
                           b1.reshape(1, 128), W2.astype(jnp.bfloat16))

    v4 = _make_prop64(NP, E)(u0, u1, u2, u3, src, dst)

    wp4 = Wp.reshape(4, 16).astype(jnp.bfloat16)
    out = _tc_c(v4, u0, u1, u2, u3, dinv, b2.reshape(1, 64), wp4,
                bp.reshape(1, 1))
    return out[:N, 0]


# speedup vs baseline: 1.1257x; 1.1257x over previous
import functools

import jax
import jax.numpy as jnp
from jax import lax
from jax.experimental import pallas as pl
from jax.experimental.pallas import tpu as pltpu
from jax.experimental.pallas import tpu_sc as plsc

NC = 2
NS = 16
NW = NC * NS
EB = 2000


def _mesh():
    return plsc.VectorSubcoreMesh(core_axis_name="c", subcore_axis_name="s")


_SC_PARAMS = pltpu.CompilerParams(use_tc_tiling_on_sc=False)


def _make_degree(NP, E):
    per_tile = E // NW
    n_blk = per_tile // EB
    z = NP // NS

    @functools.partial(
        pl.kernel,
        out_type=jax.ShapeDtypeStruct((NC * NP,), jnp.float32),
        mesh=_mesh(),
        compiler_params=_SC_PARAMS,
        scratch_types=[
            pltpu.VMEM((EB,), jnp.int32),
            pltpu.VMEM((EB,), jnp.float32),
            pltpu.VMEM((z,), jnp.float32),
            pltpu.VMEM_SHARED((NP,), jnp.float32),
            pltpu.SemaphoreType.DMA,
        ],
    )
    def deg_kernel(dst_hbm, out_hbm, didx, ones_v, stage, acc, sem):
        c = lax.axis_index("c")
        s = lax.axis_index("s")
        tile_base = (c * NS + s) * per_tile

        def set_ones(i, _):
            ones_v[pl.ds(i * 16, 16)] = jnp.full((16,), 1.0, jnp.float32)
            return 0

        lax.fori_loop(0, EB // 16, set_ones, 0)

        def set_zero(i, _):
            stage[pl.ds(i * 16, 16)] = jnp.zeros((16,), jnp.float32)
            return 0

        lax.fori_loop(0, z // 16, set_zero, 0)
        pltpu.sync_copy(stage, acc.at[pl.ds(s * z, z)])
        plsc.subcore_barrier()

        def blk(i, _):
            base = tile_base + i * EB
            pltpu.sync_copy(dst_hbm.at[pl.ds(base, EB)], didx)
            pltpu.sync_copy(ones_v, acc.at[didx], add=True)
            return 0

        lax.fori_loop(0, n_blk, blk, 0)
        plsc.subcore_barrier()
        pltpu.sync_copy(acc.at[pl.ds(s * z, z)], stage)
        pltpu.sync_copy(stage, out_hbm.at[pl.ds(c * NP + s * z, z)])

    return deg_kernel


def _make_prop16(NP, E):
    EB = 400
    per_tile = E // NW
    n_blk = per_tile // EB
    z = NP // NS

    @functools.partial(
        pl.kernel,
        out_type=jax.ShapeDtypeStruct((NC * NP, 16), jnp.float32),
        mesh=_mesh(),
        compiler_params=_SC_PARAMS,
        scratch_types=[
            pltpu.VMEM((EB,), jnp.int32),
            pltpu.VMEM((EB,), jnp.int32),
            pltpu.VMEM((EB, 16), jnp.float32),
            pltpu.VMEM_SHARED((NP, 16), jnp.float32),
            pltpu.SemaphoreType.DMA,
        ],
    )
    def prop_kernel(y_hbm, src_hbm, dst_hbm, out_hbm,
                    sidx, didx, rows, acc, sem):
        c = lax.axis_index("c")
        s = lax.axis_index("s")
        tile_base = (c * NS + s) * per_tile
        chunks = []
        off = 0
        while off < z:
            n = min(EB, z - off)
            chunks.append((off, n))
            off += n

        def fill_zero(i, _):
            rows[i] = jnp.zeros((16,), jnp.float32)
            return 0

        lax.fori_loop(0, EB, fill_zero, 0)
        for (o, n) in chunks:
            pltpu.sync_copy(rows.at[pl.ds(0, n)], acc.at[pl.ds(s * z + o, n)])
        plsc.subcore_barrier()

        def blk(i, _):
            base = tile_base + i * EB
            pltpu.sync_copy(src_hbm.at[pl.ds(base, EB)], sidx)
            pltpu.sync_copy(dst_hbm.at[pl.ds(base, EB)], didx)
            pltpu.async_copy(y_hbm.at[sidx], rows, sem).wait()
            pltpu.sync_copy(rows, acc.at[didx], add=True)
            return 0

        lax.fori_loop(0, n_blk, blk, 0)
        plsc.subcore_barrier()
        for (o, n) in chunks:
            pltpu.sync_copy(acc.at[pl.ds(s * z + o, n)], rows.at[pl.ds(0, n)])
            pltpu.sync_copy(rows.at[pl.ds(0, n)],
                            out_hbm.at[pl.ds(c * NP + s * z + o, n)])

    return prop_kernel


def _make_prop64(NP, E):
    EB = 800
    per_tile = E // NS
    n_blk = per_tile // EB
    z = NP // NS

    @functools.partial(
        pl.kernel,
        out_type=jax.ShapeDtypeStruct((4 * NP, 16), jnp.float32),
        mesh=_mesh(),
        compiler_params=_SC_PARAMS,
        scratch_types=[
            pltpu.VMEM((EB,), jnp.int32),
            pltpu.VMEM((EB,), jnp.int32),
            pltpu.VMEM((EB, 16), jnp.float32),
            pltpu.VMEM_SHARED((NP, 16), jnp.float32),
            pltpu.SemaphoreType.DMA,
        ],
    )
    def prop_kernel(u0_hbm, u1_hbm, u2_hbm, u3_hbm, src_hbm, dst_hbm,
                    out_hbm, sidx, didx, rows, acc, sem):
        c = lax.axis_index("c")
        s = lax.axis_index("s")
        tile_base = s * per_tile
        u_refs = (u0_hbm, u1_hbm, u2_hbm, u3_hbm)
        chunks = []
        off = 0
        while off < z:
            n = min(EB, z - off)
            chunks.append((off, n))
            off += n

        def fill_zero(i, _):
            rows[i] = jnp.zeros((16,), jnp.float32)
            return 0

        for r in range(2):
            q = c + 2 * r

            lax.fori_loop(0, EB, fill_zero, 0)
            for (o, n) in chunks:
                pltpu.sync_copy(rows.at[pl.ds(0, n)],
                                acc.at[pl.ds(s * z + o, n)])
            plsc.subcore_barrier()

            def blk(i, _):
                base = tile_base + i * EB
                pltpu.sync_copy(src_hbm.at[pl.ds(base, EB)], sidx)
                pltpu.sync_copy(dst_hbm.at[pl.ds(base, EB)], didx)
                for qq in range(4):
                    @pl.when(q == qq)
                    def _gather():
                        pltpu.async_copy(u_refs[qq].at[sidx], rows,
                                         sem).wait()
                pltpu.sync_copy(rows, acc.at[didx], add=True)
                return 0

            lax.fori_loop(0, n_blk, blk, 0)
            plsc.subcore_barrier()
            for (o, n) in chunks:
                pltpu.sync_copy(acc.at[pl.ds(s * z + o, n)],
                                rows.at[pl.ds(0, n)])
                pltpu.sync_copy(rows.at[pl.ds(0, n)],
                                out_hbm.at[pl.ds(q * NP + s * z + o, n)])
            plsc.subcore_barrier()

    return prop_kernel


_BN = 3128


def _tc_a(deg2, x):
    NP = x.shape[0]
    nb = NP // _BN

    def body(d0_r, d1_r, x_r, dinv_o, y_o):
        deg = d0_r[...] + d1_r[...] + 1.0
        dinv = lax.rsqrt(jnp.maximum(deg, 1.0))
        dinv_o[...] = dinv
        xb = x_r[...].astype(jnp.float32)
        y_o[...] = jnp.concatenate(
            [xb * dinv, jnp.zeros((_BN, 14), jnp.float32)], axis=1)

    return pl.pallas_call(
        body,
        grid=(nb,),
        in_specs=[
            pl.BlockSpec((_BN, 1), lambda i: (i, 0)),
            pl.BlockSpec((_BN, 1), lambda i: (i + NP // _BN, 0)),
            pl.BlockSpec((_BN, 2), lambda i: (i, 0)),
        ],
        out_specs=[
            pl.BlockSpec((_BN, 1), lambda i: (i, 0)),
            pl.BlockSpec((_BN, 16), lambda i: (i, 0)),
        ],
        out_shape=[
            jax.ShapeDtypeStruct((NP, 1), jnp.float32),
            jax.ShapeDtypeStruct((NP, 16), jnp.float32),
        ],
    )(deg2, deg2, x)


def _tc_b(zp, y16, dinv, W1, b1, W2):
    NP = y16.shape[0]
    nb = NP // _BN

    def body(z0_r, z1_r, y_r, dinv_r, w1_r, b1_r, w2_r,
             u0_o, u1_o, u2_o, u3_o):
        p = (z0_r[..., :2] + z1_r[..., :2] + y_r[..., :2]) * dinv_r[...]
        w1 = w1_r[...].astype(jnp.float32)
        h1 = jnp.maximum(
            p[:, 0:1] * w1[0:1, :] + p[:, 1:2] * w1[1:2, :] + b1_r[...], 0.0)
        t = jnp.dot(h1.astype(jnp.bfloat16), w2_r[...],
                    preferred_element_type=jnp.float32)
        u = t * dinv_r[...]
        for q, o_r in enumerate((u0_o, u1_o, u2_o, u3_o)):
            o_r[...] = u[:, 16 * q:16 * q + 16]

    qshape = jax.ShapeDtypeStruct((NP, 16), jnp.float32)
    return pl.pallas_call(
        body,
        grid=(nb,),
        in_specs=[
            pl.BlockSpec((_BN, 16), lambda i: (i, 0)),
            pl.BlockSpec((_BN, 16), lambda i: (i + NP // _BN, 0)),
            pl.BlockSpec((_BN, 16), lambda i: (i, 0)),
            pl.BlockSpec((_BN, 1), lambda i: (i, 0)),
            pl.BlockSpec((2, 128), lambda i: (0, 0)),
            pl.BlockSpec((1, 128), lambda i: (0, 0)),
            pl.BlockSpec((128, 64), lambda i: (0, 0)),
        ],
        out_specs=[pl.BlockSpec((_BN, 16), lambda i: (i, 0))] * 4,
        out_shape=[qshape] * 4,
    )(zp, zp, y16, dinv, W1, b1, W2)


def _tc_c(v4, u0, u1, u2, u3, dinv, b2, wpT, bp):
    NP = dinv.shape[0]
    nb = NP // _BN

    def qmap(q):
        return lambda i: (q * (NP // _BN) + i, 0)

    def body(v0, v1, v2, v3, u0, u1, u2, u3, dinv_r, b2_r, wp_r, bp_r, o_r):
        dinv = dinv_r[...]
        b2 = b2_r[...]
        wp = wp_r[...]
        acc = jnp.zeros((_BN, 1), jnp.float32) + bp_r[...]
        for q, (v_r, u_r) in enumerate(((v0, u0), (v1, u1), (v2, u2),
                                        (v3, u3))):
            h2 = jnp.maximum(
                (v_r[...] + u_r[...]) * dinv + b2[:, 16 * q:16 * q + 16], 0.0)
            acc = acc + jnp.dot(h2.astype(jnp.bfloat16),
                                wp[q].reshape(16, 1),
                                preferred_element_type=jnp.float32)
        o_r[...] = acc

    return pl.pallas_call(
        body,
        grid=(nb,),
        in_specs=[pl.BlockSpec((_BN, 16), qmap(q)) for q in range(4)]
        + [pl.BlockSpec((_BN, 16), lambda i: (i, 0)) for _ in range(4)]
        + [
            pl.BlockSpec((_BN, 1), lambda i: (i, 0)),
            pl.BlockSpec((1, 64), lambda i: (0, 0)),
            pl.BlockSpec((4, 16), lambda i: (0, 0)),
            pl.BlockSpec((1, 1), lambda i: (0, 0)),
        ],
        out_specs=pl.BlockSpec((_BN, 1), lambda i: (i, 0)),
        out_shape=jax.ShapeDtypeStruct((NP, 1), jnp.float32),
    )(v4, v4, v4, v4, u0, u1, u2, u3, dinv, b2, wpT, bp)


def kernel(x, edge_index, W1, b1, W2, b2, Wp, bp):
    N = x.shape[0]
    E = edge_index.shape[1]
    NP = ((N + 8 * NS - 1) // (8 * NS)) * (8 * NS)
    src = edge_index[0]
    dst = edge_index[1]

    x_p = jnp.pad(x, ((0, NP - N), (0, 0))).astype(jnp.bfloat16)

    deg2 = _make_degree(NP, E)(dst).reshape(2 * NP, 1)
    dinv, y16 = _tc_a(deg2, x_p)

    zp = _make_prop16(NP, E)(y16, src, dst)
    u0, u1, u2, u3 = _tc_b(zp, y16, dinv, W1.astype(jnp.bfloat16),
                           b1.reshape(1, 128), W2.astype(jnp.bfloat16))

    v4 = _make_prop64(NP, E)(u0, u1, u2, u3, src, dst)

    wp4 = Wp.reshape(4, 16).astype(jnp.bfloat16)
    out = _tc_c(v4, u0, u1, u2, u3, dinv, b2.reshape(1, 64), wp4,
                bp.reshape(1, 1))
    return out[:N, 0]
